# baseline (speedup 1.0000x reference)
import functools

import jax
import jax.numpy as jnp
from jax import lax
from jax.experimental import pallas as pl
from jax.experimental.pallas import tpu as pltpu
from jax.experimental.pallas import tpu_sc as plsc

_NUM_CORES = 2
_NUM_SUBCORES = 16
_NW = _NUM_CORES * _NUM_SUBCORES
_CHUNK = 128
_RING = 7


def _sc_gather(cos_cached, sin_cached, idx):
    bsz, l = idx.shape
    d = cos_cached.shape[1]
    n = bsz * l
    per_w = n // _NW
    n_chunks = per_w // _CHUNK
    nx = 2 * n_chunks
    w_per_b = _NW // bsz
    mesh = plsc.VectorSubcoreMesh(core_axis_name="c", subcore_axis_name="s")

    @functools.partial(
        pl.kernel,
        mesh=mesh,
        out_type=(
            jax.ShapeDtypeStruct((bsz, 1, l, d), jnp.float32),
            jax.ShapeDtypeStruct((bsz, 1, l, d), jnp.float32),
        ),
        scratch_types=[
            pltpu.VMEM((per_w,), jnp.int32),
            pltpu.VMEM((_RING, _CHUNK, d), jnp.float32),
        ]
        + [pltpu.SemaphoreType.DMA] * (2 * nx),
    )
    def body(cos_hbm, sin_hbm, idx_hbm, cos_out, sin_out, idx_v, bufs, *sems):
        gsem = sems[:nx]
        wsem = sems[nx:]
        wid = lax.axis_index("s") * _NUM_CORES + lax.axis_index("c")
        bb = wid // w_per_b
        ofs = (wid % w_per_b) * per_w

        xfers = []
        for j in range(n_chunks):
            xfers.append((j, cos_hbm, cos_out))
            xfers.append((j, sin_hbm, sin_out))

        def start_gather(i, b):
            j, tab, _ = xfers[i]
            return pltpu.async_copy(
                tab.at[idx_v.at[pl.ds(j * _CHUNK, _CHUNK)]], bufs.at[b],
                gsem[i])

        def start_write(i, b):
            j, _, out_hbm = xfers[i]
            return pltpu.async_copy(
                bufs.at[b],
                out_hbm.at[bb, 0, pl.ds(ofs + j * _CHUNK, _CHUNK)],
                wsem[i])

        pltpu.sync_copy(idx_hbm.at[bb, pl.ds(ofs, _CHUNK)],
                        idx_v.at[pl.ds(0, _CHUNK)])
        g = [start_gather(0, 0), start_gather(1, 1)]
        pltpu.sync_copy(idx_hbm.at[bb, pl.ds(ofs + _CHUNK, per_w - _CHUNK)],
                        idx_v.at[pl.ds(_CHUNK, per_w - _CHUNK)])
        g += [start_gather(i, i) for i in range(2, min(nx, _RING))]

        w = [None] * nx
        for i in range(nx):
            b = i % _RING
            g[b].wait()
            w[i] = start_write(i, b)
            if i + _RING < nx:
                w[i].wait()
                g[b] = start_gather(i + _RING, b)
        for i in range(max(nx - _RING, 0), nx):
            w[i].wait()

    return body(cos_cached, sin_cached, idx)


def kernel(x, position_ids, cos_cached, sin_cached):
    bsz, l = position_ids.shape
    assert (bsz * l) % (_NW * _CHUNK) == 0
    idx = position_ids.astype(jnp.int32)
    cos, sin = _sc_gather(cos_cached, sin_cached, idx)
    return cos.astype(x.dtype), sin.astype(x.dtype)


# speedup vs baseline: 1.0188x; 1.0188x over previous
import functools

import jax
import jax.numpy as jnp
from jax import lax
from jax.experimental import pallas as pl
from jax.experimental.pallas import tpu as pltpu
from jax.experimental.pallas import tpu_sc as plsc

_NUM_CORES = 2
_NUM_SUBCORES = 16
_NW = _NUM_CORES * _NUM_SUBCORES
_CHUNK = 128
_RING = 6


def _sc_gather(cos_cached, sin_cached, idx):
    bsz, l = idx.shape
    d = cos_cached.shape[1]
    n = bsz * l
    per_w = n // _NW
    n_chunks = per_w // _CHUNK
    w_per_b = _NW // bsz
    mesh = plsc.VectorSubcoreMesh(core_axis_name="c", subcore_axis_name="s")

    @functools.partial(
        pl.kernel,
        mesh=mesh,
        out_type=(
            jax.ShapeDtypeStruct((bsz, 1, l, d), jnp.float32),
            jax.ShapeDtypeStruct((bsz, 1, l, d), jnp.float32),
        ),
        scratch_types=[
            pltpu.VMEM((per_w,), jnp.int32),
            pltpu.VMEM((_RING, _CHUNK, d), jnp.float32),
        ]
        + [pltpu.SemaphoreType.DMA] * (2 * _RING),
    )
    def body(cos_hbm, sin_hbm, idx_hbm, cos_out, sin_out, idx_v, bufs, *sems):
        gsem = sems[:_RING]
        wsem = sems[_RING:]
        wid = lax.axis_index("s") * _NUM_CORES + lax.axis_index("c")
        bb = wid // w_per_b
        ofs = (wid % w_per_b) * per_w
        pltpu.sync_copy(idx_hbm.at[bb, pl.ds(ofs, per_w)], idx_v)

        xfers = []
        for j in range(n_chunks):
            xfers.append((j, cos_hbm, cos_out))
            xfers.append((j, sin_hbm, sin_out))
        nx = len(xfers)

        def start_gather(i, b):
            j, tab, _ = xfers[i]
            return pltpu.async_copy(
                tab.at[idx_v.at[pl.ds(j * _CHUNK, _CHUNK)]], bufs.at[b],
                gsem[b])

        g = [None] * _RING
        w = [None] * _RING
        for i in range(min(nx, _RING)):
            g[i] = start_gather(i, i)
        for i in range(nx):
            b = i % _RING
            j, _, out_hbm = xfers[i]
            g[b].wait()
            w[b] = pltpu.async_copy(
                bufs.at[b],
                out_hbm.at[bb, 0, pl.ds(ofs + j * _CHUNK, _CHUNK)],
                wsem[b])
            if i + _RING < nx:
                w[b].wait()
                g[b] = start_gather(i + _RING, b)
        for i in range(max(nx - _RING, 0), nx):
            w[i % _RING].wait()

    return body(cos_cached, sin_cached, idx)


def kernel(x, position_ids, cos_cached, sin_cached):
    bsz, l = position_ids.shape
    assert (bsz * l) % (_NW * _CHUNK) == 0
    idx = position_ids.astype(jnp.int32)
    cos, sin = _sc_gather(cos_cached, sin_cached, idx)
    return cos.astype(x.dtype), sin.astype(x.dtype)


# speedup vs baseline: 1.0230x; 1.0041x over previous
import functools

import jax
import jax.numpy as jnp
from jax import lax
from jax.experimental import pallas as pl
from jax.experimental.pallas import tpu as pltpu
from jax.experimental.pallas import tpu_sc as plsc

_NUM_CORES = 2
_NUM_SUBCORES = 16
_NW = _NUM_CORES * _NUM_SUBCORES
_CHUNK = 128
_RING = 6


def _sc_gather(cos_cached, sin_cached, idx):
    bsz, l = idx.shape
    d = cos_cached.shape[1]
    n = bsz * l
    per_w = n // _NW
    n_chunks = per_w // _CHUNK
    w_per_b = _NW // bsz
    mesh = plsc.VectorSubcoreMesh(core_axis_name="c", subcore_axis_name="s")

    @functools.partial(
        pl.kernel,
        mesh=mesh,
        out_type=(
            jax.ShapeDtypeStruct((bsz, 1, l, d), jnp.float32),
            jax.ShapeDtypeStruct((bsz, 1, l, d), jnp.float32),
        ),
        scratch_types=[
            pltpu.VMEM((per_w,), jnp.int32),
            pltpu.VMEM((_RING, _CHUNK, d), jnp.float32),
        ]
        + [pltpu.SemaphoreType.DMA] * (2 * _RING),
    )
    def body(cos_hbm, sin_hbm, idx_hbm, cos_out, sin_out, idx_v, bufs, *sems):
        gsem = sems[:_RING]
        wsem = sems[_RING:]
        wid = lax.axis_index("s") * _NUM_CORES + lax.axis_index("c")
        bb = wid // w_per_b
        ofs = (wid % w_per_b) * per_w
        pltpu.sync_copy(idx_hbm.at[bb, pl.ds(ofs, per_w)], idx_v)

        xfers = []
        for j in range(n_chunks):
            xfers.append((j, cos_hbm, cos_out))
        for j in range(n_chunks):
            xfers.append((j, sin_hbm, sin_out))
        nx = len(xfers)

        def start_gather(i, b):
            j, tab, _ = xfers[i]
            return pltpu.async_copy(
                tab.at[idx_v.at[pl.ds(j * _CHUNK, _CHUNK)]], bufs.at[b],
                gsem[b])

        g = [None] * _RING
        w = [None] * _RING
        for i in range(min(nx, _RING)):
            g[i] = start_gather(i, i)
        for i in range(nx):
            b = i % _RING
            j, _, out_hbm = xfers[i]
            g[b].wait()
            w[b] = pltpu.async_copy(
                bufs.at[b],
                out_hbm.at[bb, 0, pl.ds(ofs + j * _CHUNK, _CHUNK)],
                wsem[b])
            if i + _RING < nx:
                w[b].wait()
                g[b] = start_gather(i + _RING, b)
        for i in range(max(nx - _RING, 0), nx):
            w[i % _RING].wait()

    return body(cos_cached, sin_cached, idx)


def kernel(x, position_ids, cos_cached, sin_cached):
    bsz, l = position_ids.shape
    assert (bsz * l) % (_NW * _CHUNK) == 0
    idx = position_ids.astype(jnp.int32)
    cos, sin = _sc_gather(cos_cached, sin_cached, idx)
    return cos.astype(x.dtype), sin.astype(x.dtype)
